# baseline (speedup 1.0000x reference)
import functools

import jax
import jax.numpy as jnp
from jax import lax
from jax.experimental import pallas as pl
from jax.experimental.pallas import tpu as pltpu
from jax.experimental.pallas import tpu_sc as plsc

_VOCAB = 1000
_D = 128
_BATCH = 16384

_NC = 2
_NS = 16
_NW = _NC * _NS
_B2 = 2 * _BATCH
_CH = 128
_BPW = _B2 // _NW
_NCH = _BPW // _CH


def _proj_body(e_ref, w_ref, b_ref, o_ref):
    o_ref[...] = lax.dot_general(
        e_ref[...], w_ref[...],
        dimension_numbers=(((1,), (1,)), ((), ())),
        preferred_element_type=jnp.float32,
    ) + b_ref[...]


def _project_table(emb_weight, lin_w, lin_b):
    return pl.pallas_call(
        _proj_body,
        out_shape=jax.ShapeDtypeStruct((_VOCAB, _D), jnp.float32),
    )(emb_weight, lin_w, lin_b.reshape(1, _D))


_mesh = plsc.VectorSubcoreMesh(core_axis_name="c", subcore_axis_name="s")

_ORPW = _BPW // 2
_ORCH = _CH // 2


@functools.partial(
    pl.kernel,
    out_type=jax.ShapeDtypeStruct((_BATCH, 2 * _D), jnp.float32),
    mesh=_mesh,
    scratch_types=[
        pltpu.VMEM((_NCH, _CH), jnp.int32),
        pltpu.VMEM((2, _CH, _D), jnp.float32),
        pltpu.SemaphoreType.DMA,
        pltpu.SemaphoreType.DMA,
    ],
)
def _gather_rows(idx_hbm, table_hbm, out_hbm, idx_v, rows_v, sem0, sem1):
    wid = lax.axis_index("s") * _NC + lax.axis_index("c")
    base = wid * _ORPW
    for j in range(_NCH):
        pltpu.sync_copy(idx_hbm.at[pl.ds(wid * _BPW + j * _CH, _CH)],
                        idx_v.at[j])
    sems = (sem0, sem1)
    cps = [None, None]
    cps[0] = pltpu.async_copy(
        table_hbm.at[idx_v.at[0]], rows_v.at[0], sems[0])
    for j in range(_NCH):
        cur = j % 2
        nxt = (j + 1) % 2
        if j + 1 < _NCH:
            cps[nxt] = pltpu.async_copy(
                table_hbm.at[idx_v.at[j + 1]], rows_v.at[nxt], sems[nxt])
        cps[cur].wait()
        pltpu.sync_copy(rows_v.at[cur].reshape(_ORCH, 2 * _D),
                        out_hbm.at[pl.ds(base + j * _ORCH, _ORCH)])


def kernel(bandwidth, emb_weight, lin_w, lin_b):
    table = _project_table(emb_weight, lin_w, lin_b)
    idx = bandwidth.astype(jnp.int32).reshape(_B2)
    return _gather_rows(idx, table)


# speedup vs baseline: 1.3771x; 1.3771x over previous
import functools

import jax
import jax.numpy as jnp
from jax import lax
from jax.experimental import pallas as pl
from jax.experimental.pallas import tpu as pltpu
from jax.experimental.pallas import tpu_sc as plsc

_VOCAB = 1000
_D = 128
_BATCH = 16384

_NC = 2
_NS = 16
_NW = _NC * _NS
_CH = 128
_ORPW = _BATCH // _NW
_NCH = _ORPW // _CH


def _proj_body(e_ref, w_ref, b_ref, o_ref):
    o_ref[...] = lax.dot_general(
        e_ref[...], w_ref[...],
        dimension_numbers=(((1,), (1,)), ((), ())),
        preferred_element_type=jnp.float32,
    ) + b_ref[...]


def _project_table(emb_weight, lin_w, lin_b):
    return pl.pallas_call(
        _proj_body,
        out_shape=jax.ShapeDtypeStruct((_VOCAB, _D), jnp.float32),
    )(emb_weight, lin_w, lin_b.reshape(1, _D))


_mesh = plsc.VectorSubcoreMesh(core_axis_name="c", subcore_axis_name="s")


@functools.partial(
    pl.kernel,
    out_type=jax.ShapeDtypeStruct((_BATCH, 2 * _D), jnp.float32),
    mesh=_mesh,
    scratch_types=[
        pltpu.VMEM((2 * _NCH, _CH), jnp.int32),
        pltpu.VMEM((2, _CH, _D), jnp.float32),
        pltpu.SemaphoreType.DMA,
        pltpu.SemaphoreType.DMA,
    ],
)
def _gather_rows(idx_hbm, table_hbm, out_hbm, idx_v, rows_v, sem0, sem1):
    wid = lax.axis_index("s") * _NC + lax.axis_index("c")
    base = wid * _ORPW
    pltpu.sync_copy(idx_hbm.at[pl.ds(wid * 2 * _NCH, 2 * _NCH)], idx_v)
    sems = (sem0, sem1)
    cps = [None, None]
    cps[0] = pltpu.async_copy(table_hbm.at[idx_v.at[0]], rows_v.at[0], sems[0])
    for j in range(2 * _NCH):
        cur = j % 2
        nxt = (j + 1) % 2
        if j + 1 < 2 * _NCH:
            cps[nxt] = pltpu.async_copy(
                table_hbm.at[idx_v.at[j + 1]], rows_v.at[nxt], sems[nxt])
        cps[cur].wait()
        row0 = base + (j // 2) * _CH
        col0 = (j % 2) * _D
        pltpu.sync_copy(rows_v.at[cur],
                        out_hbm.at[pl.ds(row0, _CH), pl.ds(col0, _D)])


def kernel(bandwidth, emb_weight, lin_w, lin_b):
    table = _project_table(emb_weight, lin_w, lin_b)
    idx = (bandwidth.astype(jnp.int32)
           .reshape(_BATCH // _CH, _CH, 2)
           .transpose(0, 2, 1)
           .reshape(2 * _BATCH // _CH, _CH))
    return _gather_rows(idx, table)
